# baseline (speedup 1.0000x reference)
import functools

import jax
import jax.numpy as jnp
from jax import lax
from jax.experimental import pallas as pl
from jax.experimental.pallas import tpu as pltpu
from jax.experimental.pallas import tpu_sc as plsc

NUM_CORES = 2
NUM_SUBCORES = 16
NW = NUM_CORES * NUM_SUBCORES

HEAD = 4
D = 128
V = 200
L = 16
B_TOTAL = 4096 * 200
B_PER_W = B_TOTAL // NW
CB = 128
NCHUNK = B_PER_W // CB
NBUF = 4
NOUTER = NCHUNK // NBUF


def _reduce_table_body(t_ref, out_ref):
    t = t_ref[...]
    acc = t[:, 0:D] + t[:, D:2 * D] + t[:, 2 * D:3 * D] + t[:, 3 * D:4 * D]
    out_ref[...] = acc * (1.0 / HEAD)


def _reduce_table(table):
    return pl.pallas_call(
        _reduce_table_body,
        out_shape=jax.ShapeDtypeStruct((V, D), jnp.float32),
    )(table)


def _gather_body(idx_hbm, rt_hbm, out_hbm, idx_v, tab_v, rows_v, *wsems):
    wid = lax.axis_index("s") * NUM_CORES + lax.axis_index("c")
    base = wid * B_PER_W

    pltpu.sync_copy(idx_hbm.at[wid], idx_v)
    pltpu.sync_copy(rt_hbm, tab_v)

    lanes = lax.iota(jnp.int32, L)

    def compute_chunk(b, j):
        def grp(g, c0):
            idx16 = idx_v[j, pl.ds(g * L, L)]

            def col(c, carry):
                addr_t, addr_o = carry
                v = plsc.load_gather(tab_v, [addr_t])
                plsc.store_scatter(rows_v, [addr_o], v)
                return addr_t + 1, addr_o + 1

            lax.fori_loop(0, D, col,
                          (idx16 * D, (b * CB + g * L + lanes) * D),
                          unroll=4)
            return c0

        lax.fori_loop(0, CB // L, grp, 0, unroll=False)

    def start_write(b, j):
        pltpu.async_copy(rows_v.at[pl.ds(b * CB * D, CB * D)],
                         out_hbm.at[pl.ds((base + j * CB) * D, CB * D)],
                         wsems[b])

    def wait_write(b, j):
        pltpu.make_async_copy(rows_v.at[pl.ds(b * CB * D, CB * D)],
                              out_hbm.at[pl.ds((base + j * CB) * D, CB * D)],
                              wsems[b]).wait()

    def outer(g, carry):
        for b in range(NBUF):
            j = g * NBUF + b

            @pl.when(g > 0)
            def _():
                wait_write(b, j - NBUF)

            compute_chunk(b, j)
            start_write(b, j)
        return carry

    lax.fori_loop(0, NOUTER, outer, 0, unroll=False)

    for b in range(NBUF):
        wait_write(b, (NOUTER - 1) * NBUF + b)


_sc_gather = functools.partial(
    pl.kernel,
    out_type=jax.ShapeDtypeStruct((B_TOTAL * D,), jnp.float32),
    mesh=plsc.VectorSubcoreMesh(core_axis_name="c", subcore_axis_name="s"),
    scratch_types=(
        [pltpu.VMEM((NCHUNK, CB), jnp.int32),
         pltpu.VMEM((V * D,), jnp.float32),
         pltpu.VMEM((NBUF * CB * D,), jnp.float32)]
        + [pltpu.SemaphoreType.DMA] * NBUF
    ),
    compiler_params=pltpu.CompilerParams(needs_layout_passes=False),
)(_gather_body)


def kernel(indices, table):
    idx = indices.astype(jnp.int32).reshape(NW, NCHUNK, CB)
    rt = _reduce_table(table).reshape(V * D)
    out = _sc_gather(idx, rt)
    return out.reshape(indices.shape[0], indices.shape[1], D)


# speedup vs baseline: 20.2759x; 20.2759x over previous
import functools

import jax
import jax.numpy as jnp
from jax import lax
from jax.experimental import pallas as pl
from jax.experimental.pallas import tpu as pltpu
from jax.experimental.pallas import tpu_sc as plsc

NUM_CORES = 2
NUM_SUBCORES = 16
NW = NUM_CORES * NUM_SUBCORES

HEAD = 4
D = 128
V = 200
B_TOTAL = 4096 * 200
B_PER_W = B_TOTAL // NW
CB = 128
NCHUNK = B_PER_W // CB
NBUF = 4
NOUTER = NCHUNK // NBUF


def _reduce_table_body(t_ref, out_ref):
    t = t_ref[...]
    acc = t[:, 0:D] + t[:, D:2 * D] + t[:, 2 * D:3 * D] + t[:, 3 * D:4 * D]
    out_ref[...] = acc * (1.0 / HEAD)


def _reduce_table(table):
    return pl.pallas_call(
        _reduce_table_body,
        out_shape=jax.ShapeDtypeStruct((V, D), jnp.float32),
    )(table)


def _gather_body(idx_hbm, rt_hbm, out_hbm, idx_v, rows_v, tab_sh, *sems):
    gsems = sems[:NBUF]
    wsems = sems[NBUF:]
    cid = lax.axis_index("c")
    sid = lax.axis_index("s")
    wid = sid * NUM_CORES + cid
    base = wid * B_PER_W

    @pl.when(sid == 0)
    def _():
        pltpu.sync_copy(rt_hbm, tab_sh)

    pltpu.sync_copy(idx_hbm.at[wid], idx_v)
    plsc.subcore_barrier()

    def start_gather(b, j):
        pltpu.async_copy(tab_sh.at[idx_v.at[j]], rows_v.at[b], gsems[b])

    def wait_gather(b, j):
        pltpu.make_async_copy(tab_sh.at[idx_v.at[j]], rows_v.at[b],
                              gsems[b]).wait()

    def start_write(b, j):
        pltpu.async_copy(rows_v.at[b], out_hbm.at[pl.ds(base + j * CB, CB)],
                         wsems[b])

    def wait_write(b, j):
        pltpu.make_async_copy(rows_v.at[b],
                              out_hbm.at[pl.ds(base + j * CB, CB)],
                              wsems[b]).wait()

    for b in range(NBUF):
        start_gather(b, b)

    def outer(g, carry):
        for b in range(NBUF):
            j = g * NBUF + b
            wait_gather(b, j)
            start_write(b, j)
            wait_write(b, j)
            start_gather(b, j + NBUF)
        return carry

    lax.fori_loop(0, NOUTER - 1, outer, 0, unroll=False)

    for b in range(NBUF):
        j = (NOUTER - 1) * NBUF + b
        wait_gather(b, j)
        start_write(b, j)
        wait_write(b, j)


_sc_gather = functools.partial(
    pl.kernel,
    out_type=jax.ShapeDtypeStruct((B_TOTAL, D), jnp.float32),
    mesh=plsc.VectorSubcoreMesh(core_axis_name="c", subcore_axis_name="s"),
    scratch_types=(
        [pltpu.VMEM((NCHUNK, CB), jnp.int32),
         pltpu.VMEM((NBUF, CB, D), jnp.float32),
         pltpu.MemorySpace.VMEM_SHARED((V, D), jnp.float32)]
        + [pltpu.SemaphoreType.DMA] * (2 * NBUF)
    ),
)(_gather_body)


def kernel(indices, table):
    idx = indices.astype(jnp.int32).reshape(NW, NCHUNK, CB)
    rt = _reduce_table(table)
    out = _sc_gather(idx, rt)
    return out.reshape(indices.shape[0], indices.shape[1], D)
